# baseline (speedup 1.0000x reference)
import functools

import jax
import jax.numpy as jnp
from jax import lax
from jax.experimental import pallas as pl
from jax.experimental.pallas import tpu as pltpu
from jax.experimental.pallas import tpu_sc as plsc

HIDDEN = 768
COORD = 128
MAX_2D = 1024
PAD = 1
EPS = 1e-5
CHUNK = 32
LANES = 16


def _pid_body(ids_ref, out_ref):
    ids = ids_ref[...]
    s = ids.shape[1]
    mask = (ids != PAD).astype(jnp.float32)
    iu = lax.broadcasted_iota(jnp.int32, (s, s), 0)
    it = lax.broadcasted_iota(jnp.int32, (s, s), 1)
    tri = (iu <= it).astype(jnp.float32)
    cs = jax.lax.dot(mask, tri, precision=jax.lax.Precision.HIGHEST)
    out_ref[...] = cs.astype(jnp.int32) * (ids != PAD).astype(jnp.int32) + PAD


@functools.lru_cache(maxsize=None)
def _make_pid(B, S):
    return pl.pallas_call(
        _pid_body,
        out_shape=jax.ShapeDtypeStruct((B, S), jnp.int32),
    )


def _sc_gather_builder(N, n_workers):
    tok_per_w = N // n_workers
    n_chunks = tok_per_w // CHUNK
    half = 3 * CHUNK

    def body(ids_hbm, b0_hbm, b1_hbm, b2_hbm, b3_hbm,
             word_hbm, tabs_hbm,
             wout_hbm, sout_hbm,
             ids_v, b0_v, b1_v, b2_v, b3_v, cidx_v,
             wbuf0, sbuf0, wbuf1, sbuf1, gsem0, gsem1, wsem0, wsem1):
        cid = lax.axis_index("c")
        sid = lax.axis_index("s")
        wid = sid * 2 + cid
        base = wid * tok_per_w

        pltpu.sync_copy(ids_hbm.at[pl.ds(base, tok_per_w)], ids_v)
        pltpu.sync_copy(b0_hbm.at[pl.ds(base, tok_per_w)], b0_v)
        pltpu.sync_copy(b1_hbm.at[pl.ds(base, tok_per_w)], b1_v)
        pltpu.sync_copy(b2_hbm.at[pl.ds(base, tok_per_w)], b2_v)
        pltpu.sync_copy(b3_hbm.at[pl.ds(base, tok_per_w)], b3_v)
        for g in range(tok_per_w // LANES):
            sl = pl.ds(g * LANES, LANES)
            c, r = divmod(g, CHUNK // LANES)
            cbase = c * 6 * CHUNK + r * LANES
            b0 = b0_v[sl]
            b1 = b1_v[sl]
            b2 = b2_v[sl]
            b3 = b3_v[sl]
            cidx_v[pl.ds(cbase + 0 * CHUNK, LANES)] = b0
            cidx_v[pl.ds(cbase + 1 * CHUNK, LANES)] = b1 + MAX_2D
            cidx_v[pl.ds(cbase + 2 * CHUNK, LANES)] = b2
            cidx_v[pl.ds(cbase + 3 * CHUNK, LANES)] = b3 + MAX_2D
            cidx_v[pl.ds(cbase + 4 * CHUNK, LANES)] = (
                jnp.clip(b3 - b1, 0, MAX_2D - 1) + 2 * MAX_2D)
            cidx_v[pl.ds(cbase + 5 * CHUNK, LANES)] = (
                jnp.clip(b2 - b0, 0, MAX_2D - 1) + 3 * MAX_2D)

        def gather_descs(c, wbuf, sbuf, gsem):
            off = c * CHUNK
            soff = c * 6 * CHUNK
            return [
                pltpu.make_async_copy(word_hbm.at[ids_v.at[pl.ds(off, CHUNK)]], wbuf, gsem),
                pltpu.make_async_copy(tabs_hbm.at[cidx_v.at[pl.ds(soff, half)]],
                                      sbuf.at[pl.ds(0, half)], gsem),
                pltpu.make_async_copy(tabs_hbm.at[cidx_v.at[pl.ds(soff + half, half)]],
                                      sbuf.at[pl.ds(half, half)], gsem),
            ]

        def write_descs(c, wbuf, sbuf, wsem):
            tok = base + c * CHUNK
            return [
                pltpu.make_async_copy(wbuf, wout_hbm.at[pl.ds(tok, CHUNK)], wsem),
                pltpu.make_async_copy(sbuf, sout_hbm.at[pl.ds(tok * 6, 6 * CHUNK)], wsem),
            ]

        def issue(descs):
            for d in descs:
                d.start()

        def wait(descs):
            for d in descs:
                d.wait()

        issue(gather_descs(0, wbuf0, sbuf0, gsem0))
        issue(gather_descs(1, wbuf1, sbuf1, gsem1))
        wait(gather_descs(0, wbuf0, sbuf0, gsem0))
        issue(write_descs(0, wbuf0, sbuf0, wsem0))

        def pipe_body(i, carry):
            c0 = 2 * i
            wait(write_descs(c0 - 2, wbuf0, sbuf0, wsem0))
            issue(gather_descs(c0, wbuf0, sbuf0, gsem0))
            wait(gather_descs(c0 - 1, wbuf1, sbuf1, gsem1))
            issue(write_descs(c0 - 1, wbuf1, sbuf1, wsem1))
            wait(write_descs(c0 - 1, wbuf1, sbuf1, wsem1))
            issue(gather_descs(c0 + 1, wbuf1, sbuf1, gsem1))
            wait(gather_descs(c0, wbuf0, sbuf0, gsem0))
            issue(write_descs(c0, wbuf0, sbuf0, wsem0))
            return carry

        lax.fori_loop(1, n_chunks // 2, pipe_body, jnp.int32(0))

        wait(gather_descs(n_chunks - 1, wbuf1, sbuf1, gsem1))
        issue(write_descs(n_chunks - 1, wbuf1, sbuf1, wsem1))
        wait(write_descs(n_chunks - 2, wbuf0, sbuf0, wsem0))
        wait(write_descs(n_chunks - 1, wbuf1, sbuf1, wsem1))

    return body


@functools.lru_cache(maxsize=None)
def _make_sc_gather(N):
    info = plsc.get_sparse_core_info()
    n_workers = info.num_cores * info.num_subcores
    tok_per_w = N // n_workers
    mesh = plsc.VectorSubcoreMesh(core_axis_name="c", subcore_axis_name="s")
    return pl.kernel(
        _sc_gather_builder(N, n_workers),
        out_type=(jax.ShapeDtypeStruct((N, HIDDEN), jnp.float32),
                  jax.ShapeDtypeStruct((6 * N, COORD), jnp.float32)),
        mesh=mesh,
        scratch_types=[
            pltpu.VMEM((tok_per_w,), jnp.int32),
            pltpu.VMEM((tok_per_w,), jnp.int32),
            pltpu.VMEM((tok_per_w,), jnp.int32),
            pltpu.VMEM((tok_per_w,), jnp.int32),
            pltpu.VMEM((tok_per_w,), jnp.int32),
            pltpu.VMEM((6 * tok_per_w,), jnp.int32),
            pltpu.VMEM((CHUNK, HIDDEN), jnp.float32),
            pltpu.VMEM((6 * CHUNK, COORD), jnp.float32),
            pltpu.VMEM((CHUNK, HIDDEN), jnp.float32),
            pltpu.VMEM((6 * CHUNK, COORD), jnp.float32),
            pltpu.SemaphoreType.DMA,
            pltpu.SemaphoreType.DMA,
            pltpu.SemaphoreType.DMA,
            pltpu.SemaphoreType.DMA,
        ],
    )


def _ln_body(w_ref, s_ref, pid_ref, pos_ref, tte_ref, g_ref, b_ref, out_ref):
    blk = w_ref.shape[0]
    n_pos = pos_ref.shape[0]
    pid_col = jnp.swapaxes(pid_ref[0], 0, 1)
    onehot = (pid_col == lax.broadcasted_iota(jnp.int32, (blk, n_pos), 1))
    pos_rows = jax.lax.dot(onehot.astype(jnp.bfloat16), pos_ref[...],
                           preferred_element_type=jnp.float32)
    x = w_ref[...] + s_ref[...] + pos_rows + tte_ref[...]
    mu = jnp.mean(x, axis=-1, keepdims=True)
    xc = x - mu
    var = jnp.mean(xc * xc, axis=-1, keepdims=True)
    out_ref[...] = xc * lax.rsqrt(var + EPS) * g_ref[...] + b_ref[...]


@functools.lru_cache(maxsize=None)
def _make_ln(N, blk, n_pos):
    return pl.pallas_call(
        _ln_body,
        grid=(N // blk,),
        in_specs=[
            pl.BlockSpec((blk, HIDDEN), lambda i: (i, 0)),
            pl.BlockSpec((blk, HIDDEN), lambda i: (i, 0)),
            pl.BlockSpec((1, 1, blk), lambda i: (i, 0, 0)),
            pl.BlockSpec((n_pos, HIDDEN), lambda i: (0, 0)),
            pl.BlockSpec((1, HIDDEN), lambda i: (0, 0)),
            pl.BlockSpec((1, HIDDEN), lambda i: (0, 0)),
            pl.BlockSpec((1, HIDDEN), lambda i: (0, 0)),
        ],
        out_specs=pl.BlockSpec((blk, HIDDEN), lambda i: (i, 0)),
        out_shape=jax.ShapeDtypeStruct((N, HIDDEN), jnp.float32),
    )


def kernel(input_ids, bbox, word_emb, token_type_emb, pos_emb,
           x_emb, y_emb, h_emb, w_emb, ln_gamma, ln_beta):
    B, S = input_ids.shape
    N = B * S
    n_pos = pos_emb.shape[0]
    pid = _make_pid(B, S)(input_ids)
    ids = input_ids.reshape(N)
    bb = bbox.reshape(N, 4)
    tabs = jnp.concatenate([x_emb, y_emb, h_emb, w_emb], axis=0)
    wrows, srowsp = _make_sc_gather(N)(
        ids, bb[:, 0], bb[:, 1], bb[:, 2], bb[:, 3], word_emb, tabs)
    srows = (srowsp.reshape(N // CHUNK, 6, CHUNK, COORD)
             .transpose(0, 2, 1, 3).reshape(N, HIDDEN))
    out = _make_ln(N, 512, n_pos)(
        wrows, srows, pid.reshape(N // 512, 1, 512), pos_emb.astype(jnp.bfloat16),
        token_type_emb, ln_gamma.reshape(1, HIDDEN), ln_beta.reshape(1, HIDDEN))
    return out.reshape(B, S, HIDDEN)


# speedup vs baseline: 4.5236x; 4.5236x over previous
import functools

import jax
import jax.numpy as jnp
from jax import lax
from jax.experimental import pallas as pl
from jax.experimental.pallas import tpu as pltpu
from jax.experimental.pallas import tpu_sc as plsc

HIDDEN = 768
COORD = 128
MAX_2D = 1024
PAD = 1
EPS = 1e-5
CHUNK = 64
LANES = 16


def _pid_body(ids_ref, out_ref):
    ids = ids_ref[...]
    s = ids.shape[1]
    mask = (ids != PAD).astype(jnp.float32)
    iu = lax.broadcasted_iota(jnp.int32, (s, s), 0)
    it = lax.broadcasted_iota(jnp.int32, (s, s), 1)
    tri = (iu <= it).astype(jnp.float32)
    cs = jax.lax.dot(mask, tri, precision=jax.lax.Precision.HIGHEST)
    out_ref[...] = cs.astype(jnp.int32) * (ids != PAD).astype(jnp.int32) + PAD


@functools.lru_cache(maxsize=None)
def _make_pid(B, S):
    return pl.pallas_call(
        _pid_body,
        out_shape=jax.ShapeDtypeStruct((B, S), jnp.int32),
    )


def _sc_gather_builder(N, n_workers):
    tok_per_w = N // n_workers
    n_chunks = tok_per_w // CHUNK

    def body(ids_hbm, word_hbm, wout_hbm,
             ids_v, wbuf0, wbuf1, gsem0, gsem1, wsem0, wsem1):
        cid = lax.axis_index("c")
        sid = lax.axis_index("s")
        wid = sid * 2 + cid
        base = wid * tok_per_w

        pltpu.sync_copy(ids_hbm.at[pl.ds(base, tok_per_w)], ids_v)

        def gather_desc(c, wbuf, gsem):
            return pltpu.make_async_copy(
                word_hbm.at[ids_v.at[pl.ds(c * CHUNK, CHUNK)]], wbuf, gsem)

        def write_desc(c, wbuf, wsem):
            return pltpu.make_async_copy(
                wbuf, wout_hbm.at[pl.ds(base + c * CHUNK, CHUNK)], wsem)

        gather_desc(0, wbuf0, gsem0).start()
        gather_desc(1, wbuf1, gsem1).start()
        gather_desc(0, wbuf0, gsem0).wait()
        write_desc(0, wbuf0, wsem0).start()

        def pipe_body(i, carry):
            c0 = 2 * i
            write_desc(c0 - 2, wbuf0, wsem0).wait()
            gather_desc(c0, wbuf0, gsem0).start()
            gather_desc(c0 - 1, wbuf1, gsem1).wait()
            write_desc(c0 - 1, wbuf1, wsem1).start()
            write_desc(c0 - 1, wbuf1, wsem1).wait()
            gather_desc(c0 + 1, wbuf1, gsem1).start()
            gather_desc(c0, wbuf0, gsem0).wait()
            write_desc(c0, wbuf0, wsem0).start()
            return carry

        lax.fori_loop(1, n_chunks // 2, pipe_body, jnp.int32(0))

        gather_desc(n_chunks - 1, wbuf1, gsem1).wait()
        write_desc(n_chunks - 1, wbuf1, wsem1).start()
        write_desc(n_chunks - 2, wbuf0, wsem0).wait()
        write_desc(n_chunks - 1, wbuf1, wsem1).wait()

    return body


@functools.lru_cache(maxsize=None)
def _make_sc_gather(N):
    info = plsc.get_sparse_core_info()
    n_workers = info.num_cores * info.num_subcores
    tok_per_w = N // n_workers
    mesh = plsc.VectorSubcoreMesh(core_axis_name="c", subcore_axis_name="s")
    return pl.kernel(
        _sc_gather_builder(N, n_workers),
        out_type=jax.ShapeDtypeStruct((N, HIDDEN), jnp.float32),
        mesh=mesh,
        scratch_types=[
            pltpu.VMEM((tok_per_w,), jnp.int32),
            pltpu.VMEM((CHUNK, HIDDEN), jnp.float32),
            pltpu.VMEM((CHUNK, HIDDEN), jnp.float32),
            pltpu.SemaphoreType.DMA,
            pltpu.SemaphoreType.DMA,
            pltpu.SemaphoreType.DMA,
            pltpu.SemaphoreType.DMA,
        ],
    )


def _onehot_rows(col, n_rows, tab_ref):
    blk = col.shape[0]
    oh = (col == lax.broadcasted_iota(jnp.int32, (blk, n_rows), 1))
    return jax.lax.dot(oh.astype(jnp.bfloat16), tab_ref[...],
                       preferred_element_type=jnp.float32)


def _ln_body(w_ref, pid_ref, b0_ref, b1_ref, b2_ref, b3_ref,
             pos_ref, x_ref, y_ref, h_ref, w2_ref,
             tte_ref, g_ref, b_ref, out_ref):
    n_pos = pos_ref.shape[0]
    pid_col = jnp.swapaxes(pid_ref[0], 0, 1)
    b0 = jnp.swapaxes(b0_ref[0], 0, 1)
    b1 = jnp.swapaxes(b1_ref[0], 0, 1)
    b2 = jnp.swapaxes(b2_ref[0], 0, 1)
    b3 = jnp.swapaxes(b3_ref[0], 0, 1)
    hi = jnp.clip(b3 - b1, 0, MAX_2D - 1)
    wi = jnp.clip(b2 - b0, 0, MAX_2D - 1)
    pos_rows = _onehot_rows(pid_col, n_pos, pos_ref)
    spatial = jnp.concatenate(
        [_onehot_rows(b0, MAX_2D, x_ref),
         _onehot_rows(b1, MAX_2D, y_ref),
         _onehot_rows(b2, MAX_2D, x_ref),
         _onehot_rows(b3, MAX_2D, y_ref),
         _onehot_rows(hi, MAX_2D, h_ref),
         _onehot_rows(wi, MAX_2D, w2_ref)], axis=-1)
    x = w_ref[...] + spatial + pos_rows + tte_ref[...]
    mu = jnp.mean(x, axis=-1, keepdims=True)
    xc = x - mu
    var = jnp.mean(xc * xc, axis=-1, keepdims=True)
    out_ref[...] = xc * lax.rsqrt(var + EPS) * g_ref[...] + b_ref[...]


@functools.lru_cache(maxsize=None)
def _make_ln(N, blk, n_pos):
    idx_spec = pl.BlockSpec((1, 1, blk), lambda i: (i, 0, 0))

    def tab_spec():
        return pl.BlockSpec((MAX_2D, COORD), lambda i: (0, 0))

    return pl.pallas_call(
        _ln_body,
        grid=(N // blk,),
        in_specs=[
            pl.BlockSpec((blk, HIDDEN), lambda i: (i, 0)),
            idx_spec, idx_spec, idx_spec, idx_spec, idx_spec,
            pl.BlockSpec((n_pos, HIDDEN), lambda i: (0, 0)),
            tab_spec(), tab_spec(), tab_spec(), tab_spec(),
            pl.BlockSpec((1, HIDDEN), lambda i: (0, 0)),
            pl.BlockSpec((1, HIDDEN), lambda i: (0, 0)),
            pl.BlockSpec((1, HIDDEN), lambda i: (0, 0)),
        ],
        out_specs=pl.BlockSpec((blk, HIDDEN), lambda i: (i, 0)),
        out_shape=jax.ShapeDtypeStruct((N, HIDDEN), jnp.float32),
    )


def kernel(input_ids, bbox, word_emb, token_type_emb, pos_emb,
           x_emb, y_emb, h_emb, w_emb, ln_gamma, ln_beta):
    B, S = input_ids.shape
    N = B * S
    n_pos = pos_emb.shape[0]
    blk = 512
    pid = _make_pid(B, S)(input_ids)
    ids = input_ids.reshape(N)
    wrows = _make_sc_gather(N)(ids, word_emb)
    bb3 = bbox.reshape(N // blk, blk, 4).transpose(0, 2, 1)
    bf16 = jnp.bfloat16
    out = _make_ln(N, blk, n_pos)(
        wrows,
        pid.reshape(N // blk, 1, blk),
        bb3[:, 0:1, :], bb3[:, 1:2, :], bb3[:, 2:3, :], bb3[:, 3:4, :],
        pos_emb.astype(bf16),
        x_emb.astype(bf16), y_emb.astype(bf16),
        h_emb.astype(bf16), w_emb.astype(bf16),
        token_type_emb, ln_gamma.reshape(1, HIDDEN), ln_beta.reshape(1, HIDDEN))
    return out.reshape(B, S, HIDDEN)


# speedup vs baseline: 4.7119x; 1.0416x over previous
import functools

import jax
import jax.numpy as jnp
from jax import lax
from jax.experimental import pallas as pl
from jax.experimental.pallas import tpu as pltpu
from jax.experimental.pallas import tpu_sc as plsc

HIDDEN = 768
COORD = 128
MAX_2D = 1024
PAD = 1
EPS = 1e-5
CHUNK = 64
LANES = 16


def _pid_body(ids_ref, out_ref):
    ids = ids_ref[...]
    s = ids.shape[1]
    mask = (ids != PAD).astype(jnp.float32)
    iu = lax.broadcasted_iota(jnp.int32, (s, s), 0)
    it = lax.broadcasted_iota(jnp.int32, (s, s), 1)
    tri = (iu <= it).astype(jnp.float32)
    cs = jax.lax.dot(mask, tri, precision=jax.lax.Precision.HIGHEST)
    out_ref[...] = cs.astype(jnp.int32) * (ids != PAD).astype(jnp.int32) + PAD


@functools.lru_cache(maxsize=None)
def _make_pid(B, S):
    return pl.pallas_call(
        _pid_body,
        out_shape=jax.ShapeDtypeStruct((B, S), jnp.int32),
    )


def _sc_gather_builder(N, n_workers):
    tok_per_w = N // n_workers
    n_chunks = tok_per_w // CHUNK

    def body(ids_hbm, word_hbm, wout_hbm,
             ids_v, wbuf0, wbuf1, gsem0, gsem1, wsem0, wsem1):
        cid = lax.axis_index("c")
        sid = lax.axis_index("s")
        wid = sid * 2 + cid
        base = wid * tok_per_w

        pltpu.sync_copy(ids_hbm.at[pl.ds(base, tok_per_w)], ids_v)

        def gather_desc(c, wbuf, gsem):
            return pltpu.make_async_copy(
                word_hbm.at[ids_v.at[pl.ds(c * CHUNK, CHUNK)]], wbuf, gsem)

        def write_desc(c, wbuf, wsem):
            return pltpu.make_async_copy(
                wbuf, wout_hbm.at[pl.ds(base + c * CHUNK, CHUNK)], wsem)

        gather_desc(0, wbuf0, gsem0).start()
        gather_desc(1, wbuf1, gsem1).start()
        gather_desc(0, wbuf0, gsem0).wait()
        write_desc(0, wbuf0, wsem0).start()

        def pipe_body(i, carry):
            c0 = 2 * i
            write_desc(c0 - 2, wbuf0, wsem0).wait()
            gather_desc(c0, wbuf0, gsem0).start()
            gather_desc(c0 - 1, wbuf1, gsem1).wait()
            write_desc(c0 - 1, wbuf1, wsem1).start()
            write_desc(c0 - 1, wbuf1, wsem1).wait()
            gather_desc(c0 + 1, wbuf1, gsem1).start()
            gather_desc(c0, wbuf0, gsem0).wait()
            write_desc(c0, wbuf0, wsem0).start()
            return carry

        lax.fori_loop(1, n_chunks // 2, pipe_body, jnp.int32(0))

        gather_desc(n_chunks - 1, wbuf1, gsem1).wait()
        write_desc(n_chunks - 1, wbuf1, wsem1).start()
        write_desc(n_chunks - 2, wbuf0, wsem0).wait()
        write_desc(n_chunks - 1, wbuf1, wsem1).wait()

    return body


@functools.lru_cache(maxsize=None)
def _make_sc_gather(N):
    info = plsc.get_sparse_core_info()
    n_workers = info.num_cores * info.num_subcores
    tok_per_w = N // n_workers
    mesh = plsc.VectorSubcoreMesh(core_axis_name="c", subcore_axis_name="s")
    return pl.kernel(
        _sc_gather_builder(N, n_workers),
        out_type=jax.ShapeDtypeStruct((N, HIDDEN), jnp.float32),
        mesh=mesh,
        scratch_types=[
            pltpu.VMEM((tok_per_w,), jnp.int32),
            pltpu.VMEM((CHUNK, HIDDEN), jnp.float32),
            pltpu.VMEM((CHUNK, HIDDEN), jnp.float32),
            pltpu.SemaphoreType.DMA,
            pltpu.SemaphoreType.DMA,
            pltpu.SemaphoreType.DMA,
            pltpu.SemaphoreType.DMA,
        ],
    )


def _onehot_rows(col, n_rows, tab_ref):
    blk = col.shape[0]
    oh = (col == lax.broadcasted_iota(jnp.int32, (blk, n_rows), 1))
    return jax.lax.dot(oh.astype(jnp.bfloat16), tab_ref[...],
                       preferred_element_type=jnp.float32)


def _ln_body(w_ref, pid_ref, b0_ref, b1_ref, b2_ref, b3_ref,
             pos_ref, x_ref, y_ref, h_ref, w2_ref,
             tte_ref, g_ref, b_ref, out_ref):
    n_pos = pos_ref.shape[0]
    pid_col = jnp.swapaxes(pid_ref[0], 0, 1)
    b0 = jnp.swapaxes(b0_ref[0], 0, 1)
    b1 = jnp.swapaxes(b1_ref[0], 0, 1)
    b2 = jnp.swapaxes(b2_ref[0], 0, 1)
    b3 = jnp.swapaxes(b3_ref[0], 0, 1)
    hi = jnp.clip(b3 - b1, 0, MAX_2D - 1)
    wi = jnp.clip(b2 - b0, 0, MAX_2D - 1)
    pos_rows = _onehot_rows(pid_col, n_pos, pos_ref)
    spatial = jnp.concatenate(
        [_onehot_rows(b0, MAX_2D, x_ref),
         _onehot_rows(b1, MAX_2D, y_ref),
         _onehot_rows(b2, MAX_2D, x_ref),
         _onehot_rows(b3, MAX_2D, y_ref),
         _onehot_rows(hi, MAX_2D, h_ref),
         _onehot_rows(wi, MAX_2D, w2_ref)], axis=-1)
    x = w_ref[...] + spatial + pos_rows + tte_ref[...]
    mu = jnp.mean(x, axis=-1, keepdims=True)
    xc = x - mu
    var = jnp.mean(xc * xc, axis=-1, keepdims=True)
    out_ref[...] = xc * lax.rsqrt(var + EPS) * g_ref[...] + b_ref[...]


@functools.lru_cache(maxsize=None)
def _make_ln(N, blk, n_pos):
    idx_spec = pl.BlockSpec((1, 1, blk), lambda i: (i, 0, 0))

    def tab_spec():
        return pl.BlockSpec((MAX_2D, COORD), lambda i: (0, 0))

    return pl.pallas_call(
        _ln_body,
        grid=(N // blk,),
        in_specs=[
            pl.BlockSpec((blk, HIDDEN), lambda i: (i, 0)),
            idx_spec, idx_spec, idx_spec, idx_spec, idx_spec,
            pl.BlockSpec((n_pos, HIDDEN), lambda i: (0, 0)),
            tab_spec(), tab_spec(), tab_spec(), tab_spec(),
            pl.BlockSpec((1, HIDDEN), lambda i: (0, 0)),
            pl.BlockSpec((1, HIDDEN), lambda i: (0, 0)),
            pl.BlockSpec((1, HIDDEN), lambda i: (0, 0)),
        ],
        out_specs=pl.BlockSpec((blk, HIDDEN), lambda i: (i, 0)),
        out_shape=jax.ShapeDtypeStruct((N, HIDDEN), jnp.float32),
    )


def kernel(input_ids, bbox, word_emb, token_type_emb, pos_emb,
           x_emb, y_emb, h_emb, w_emb, ln_gamma, ln_beta):
    B, S = input_ids.shape
    N = B * S
    n_pos = pos_emb.shape[0]
    blk = 1024
    pid = _make_pid(B, S)(input_ids)
    ids = input_ids.reshape(N)
    wrows = _make_sc_gather(N)(ids, word_emb)
    bb3 = bbox.reshape(N // blk, blk, 4).transpose(0, 2, 1)
    bf16 = jnp.bfloat16
    out = _make_ln(N, blk, n_pos)(
        wrows,
        pid.reshape(N // blk, 1, blk),
        bb3[:, 0:1, :], bb3[:, 1:2, :], bb3[:, 2:3, :], bb3[:, 3:4, :],
        pos_emb.astype(bf16),
        x_emb.astype(bf16), y_emb.astype(bf16),
        h_emb.astype(bf16), w_emb.astype(bf16),
        token_type_emb, ln_gamma.reshape(1, HIDDEN), ln_beta.reshape(1, HIDDEN))
    return out.reshape(B, S, HIDDEN)


# speedup vs baseline: 4.7829x; 1.0151x over previous
import functools

import jax
import jax.numpy as jnp
from jax import lax
from jax.experimental import pallas as pl
from jax.experimental.pallas import tpu as pltpu
from jax.experimental.pallas import tpu_sc as plsc

HIDDEN = 768
COORD = 128
MAX_2D = 1024
PAD = 1
EPS = 1e-5
CHUNK = 64
LANES = 16


def _pid_body(ids_ref, out_ref):
    ids = ids_ref[...]
    s = ids.shape[1]
    mask = (ids != PAD).astype(jnp.float32)
    iu = lax.broadcasted_iota(jnp.int32, (s, s), 0)
    it = lax.broadcasted_iota(jnp.int32, (s, s), 1)
    tri = (iu <= it).astype(jnp.float32)
    cs = jax.lax.dot(mask, tri, precision=jax.lax.Precision.HIGHEST)
    out_ref[...] = cs.astype(jnp.int32) * (ids != PAD).astype(jnp.int32) + PAD


@functools.lru_cache(maxsize=None)
def _make_pid(B, S):
    return pl.pallas_call(
        _pid_body,
        out_shape=jax.ShapeDtypeStruct((B, S), jnp.int32),
    )


def _sc_gather_builder(N, n_workers):
    tok_per_w = N // n_workers
    n_chunks = tok_per_w // CHUNK

    def body(ids_hbm, word_hbm, wout_hbm,
             ids_v, wbuf0, wbuf1, gsem0, gsem1, wsem0, wsem1):
        cid = lax.axis_index("c")
        sid = lax.axis_index("s")
        wid = sid * 2 + cid
        base = wid * tok_per_w

        pltpu.sync_copy(ids_hbm.at[pl.ds(base, tok_per_w)], ids_v)

        def gather_desc(c, wbuf, gsem):
            return pltpu.make_async_copy(
                word_hbm.at[ids_v.at[pl.ds(c * CHUNK, CHUNK)]], wbuf, gsem)

        def write_desc(c, wbuf, wsem):
            return pltpu.make_async_copy(
                wbuf, wout_hbm.at[pl.ds(base + c * CHUNK, CHUNK)], wsem)

        gather_desc(0, wbuf0, gsem0).start()
        gather_desc(1, wbuf1, gsem1).start()
        gather_desc(0, wbuf0, gsem0).wait()
        write_desc(0, wbuf0, wsem0).start()

        def pipe_body(i, carry):
            c0 = 2 * i
            write_desc(c0 - 2, wbuf0, wsem0).wait()
            gather_desc(c0, wbuf0, gsem0).start()
            gather_desc(c0 - 1, wbuf1, gsem1).wait()
            write_desc(c0 - 1, wbuf1, wsem1).start()
            write_desc(c0 - 1, wbuf1, wsem1).wait()
            gather_desc(c0 + 1, wbuf1, gsem1).start()
            gather_desc(c0, wbuf0, gsem0).wait()
            write_desc(c0, wbuf0, wsem0).start()
            return carry

        lax.fori_loop(1, n_chunks // 2, pipe_body, jnp.int32(0))

        gather_desc(n_chunks - 1, wbuf1, gsem1).wait()
        write_desc(n_chunks - 1, wbuf1, wsem1).start()
        write_desc(n_chunks - 2, wbuf0, wsem0).wait()
        write_desc(n_chunks - 1, wbuf1, wsem1).wait()

    return body


@functools.lru_cache(maxsize=None)
def _make_sc_gather(N):
    info = plsc.get_sparse_core_info()
    n_workers = info.num_cores * info.num_subcores
    tok_per_w = N // n_workers
    mesh = plsc.VectorSubcoreMesh(core_axis_name="c", subcore_axis_name="s")
    return pl.kernel(
        _sc_gather_builder(N, n_workers),
        out_type=jax.ShapeDtypeStruct((N, HIDDEN), jnp.float32),
        mesh=mesh,
        scratch_types=[
            pltpu.VMEM((tok_per_w,), jnp.int32),
            pltpu.VMEM((CHUNK, HIDDEN), jnp.float32),
            pltpu.VMEM((CHUNK, HIDDEN), jnp.float32),
            pltpu.SemaphoreType.DMA,
            pltpu.SemaphoreType.DMA,
            pltpu.SemaphoreType.DMA,
            pltpu.SemaphoreType.DMA,
        ],
    )


def _onehot_rows(col, n_rows, tab_ref):
    blk = col.shape[0]
    oh = (col == lax.broadcasted_iota(jnp.int32, (blk, n_rows), 1))
    return jax.lax.dot(oh.astype(jnp.bfloat16), tab_ref[...],
                       preferred_element_type=jnp.float32)


def _ln_body(w_ref, pid_ref, b0_ref, b1_ref, b2_ref, b3_ref,
             pos_ref, x_ref, y_ref, h_ref, w2_ref,
             tte_ref, g_ref, b_ref, out_ref):
    n_pos = pos_ref.shape[0]
    pid_col = jnp.swapaxes(pid_ref[0], 0, 1)
    b0 = jnp.swapaxes(b0_ref[0], 0, 1)
    b1 = jnp.swapaxes(b1_ref[0], 0, 1)
    b2 = jnp.swapaxes(b2_ref[0], 0, 1)
    b3 = jnp.swapaxes(b3_ref[0], 0, 1)
    hi = jnp.clip(b3 - b1, 0, MAX_2D - 1)
    wi = jnp.clip(b2 - b0, 0, MAX_2D - 1)
    pos_rows = _onehot_rows(pid_col, n_pos, pos_ref)
    spatial = jnp.concatenate(
        [_onehot_rows(b0, MAX_2D, x_ref),
         _onehot_rows(b1, MAX_2D, y_ref),
         _onehot_rows(b2, MAX_2D, x_ref),
         _onehot_rows(b3, MAX_2D, y_ref),
         _onehot_rows(hi, MAX_2D, h_ref),
         _onehot_rows(wi, MAX_2D, w2_ref)], axis=-1)
    x = w_ref[...] + spatial + pos_rows + tte_ref[...]
    mu = jnp.mean(x, axis=-1, keepdims=True)
    xc = x - mu
    var = jnp.mean(xc * xc, axis=-1, keepdims=True)
    out_ref[...] = xc * lax.rsqrt(var + EPS) * g_ref[...] + b_ref[...]


@functools.lru_cache(maxsize=None)
def _make_ln(N, blk, n_pos):
    idx_spec = pl.BlockSpec((1, 1, blk), lambda i: (i, 0, 0))

    def tab_spec():
        return pl.BlockSpec((MAX_2D, COORD), lambda i: (0, 0))

    return pl.pallas_call(
        _ln_body,
        grid=(N // blk,),
        in_specs=[
            pl.BlockSpec((blk, HIDDEN), lambda i: (i, 0)),
            idx_spec, idx_spec, idx_spec, idx_spec, idx_spec,
            pl.BlockSpec((n_pos, HIDDEN), lambda i: (0, 0)),
            tab_spec(), tab_spec(), tab_spec(), tab_spec(),
            pl.BlockSpec((1, HIDDEN), lambda i: (0, 0)),
            pl.BlockSpec((1, HIDDEN), lambda i: (0, 0)),
            pl.BlockSpec((1, HIDDEN), lambda i: (0, 0)),
        ],
        out_specs=pl.BlockSpec((blk, HIDDEN), lambda i: (i, 0)),
        out_shape=jax.ShapeDtypeStruct((N, HIDDEN), jnp.float32),
    )


def kernel(input_ids, bbox, word_emb, token_type_emb, pos_emb,
           x_emb, y_emb, h_emb, w_emb, ln_gamma, ln_beta):
    B, S = input_ids.shape
    N = B * S
    n_pos = pos_emb.shape[0]
    blk = 2048
    pid = _make_pid(B, S)(input_ids)
    ids = input_ids.reshape(N)
    wrows = _make_sc_gather(N)(ids, word_emb)
    bb3 = bbox.reshape(N // blk, blk, 4).transpose(0, 2, 1)
    bf16 = jnp.bfloat16
    out = _make_ln(N, blk, n_pos)(
        wrows,
        pid.reshape(N // blk, 1, blk),
        bb3[:, 0:1, :], bb3[:, 1:2, :], bb3[:, 2:3, :], bb3[:, 3:4, :],
        pos_emb.astype(bf16),
        x_emb.astype(bf16), y_emb.astype(bf16),
        h_emb.astype(bf16), w_emb.astype(bf16),
        token_type_emb, ln_gamma.reshape(1, HIDDEN), ln_beta.reshape(1, HIDDEN))
    return out.reshape(B, S, HIDDEN)
